# baseline (speedup 1.0000x reference)
import functools

import jax
import jax.numpy as jnp
from jax import lax
from jax.experimental import pallas as pl
from jax.experimental.pallas import tpu as pltpu
from jax.experimental.pallas import tpu_sc as plsc

MAX_H = 64
MAX_W = 64

NC = 2
NS = 16
NW = NC * NS

L = 16
CHW = 2048
NRING = 4


def _make_sc_kernel(B, H, W, C):
    total = B * H * W * C
    per_batch = H * W * C
    seg = total // NW
    nchunk = seg // CHW
    row_w = W * C
    table_row_w = MAX_W * C

    mesh = plsc.VectorSubcoreMesh(core_axis_name="c", subcore_axis_name="s")

    @functools.partial(
        pl.kernel,
        mesh=mesh,
        out_type=jax.ShapeDtypeStruct((total,), jnp.float32),
        scratch_types=[
            pltpu.VMEM((per_batch,), jnp.float32),
            pltpu.VMEM((NRING, CHW), jnp.float32),
            pltpu.VMEM((NRING, CHW), jnp.float32),
            pltpu.SemaphoreType.DMA,
            pltpu.SemaphoreType.DMA,
        ],
    )
    def sc_kernel(x_hbm, pt_hbm, o_hbm, pos_v, in_b, out_b, in_sem, out_sem):
        wid = lax.axis_index("s") * NC + lax.axis_index("c")
        base = wid * seg

        def load_pos(h, carry):
            pltpu.sync_copy(
                pt_hbm.at[pl.ds(h * table_row_w, row_w)],
                pos_v.at[pl.ds(h * row_w, row_w)],
            )
            return carry

        lax.fori_loop(0, H, load_pos, 0)

        def start_in(c, slot):
            pltpu.make_async_copy(
                x_hbm.at[pl.ds(base + c * CHW, CHW)], in_b.at[slot], in_sem
            ).start()

        for s in range(NRING):
            start_in(s, s)

        def add_chunk(c, slot):
            off = lax.rem(c * CHW, per_batch)

            def j_body(j, carry):
                out_b[slot, pl.ds(j * L, L)] = (
                    in_b[slot, pl.ds(j * L, L)] + pos_v[pl.ds(off + j * L, L)]
                )
                return carry

            lax.fori_loop(0, CHW // L, j_body, 0)

        def group(g, carry):
            for s in range(NRING):
                c = g * NRING + s
                pltpu.make_async_copy(
                    x_hbm.at[pl.ds(base + c * CHW, CHW)], in_b.at[s], in_sem
                ).wait()

                @pl.when(g >= 1)
                def _():
                    pltpu.make_async_copy(
                        out_b.at[s],
                        o_hbm.at[pl.ds(base + (c - NRING) * CHW, CHW)],
                        out_sem,
                    ).wait()

                add_chunk(c, s)

                pltpu.make_async_copy(
                    out_b.at[s], o_hbm.at[pl.ds(base + c * CHW, CHW)], out_sem
                ).start()

                @pl.when(c + NRING < nchunk)
                def _():
                    start_in(c + NRING, s)

            return carry

        lax.fori_loop(0, nchunk // NRING, group, 0)

        for s in range(NRING):
            pltpu.make_async_copy(
                out_b.at[s], o_hbm.at[pl.ds(base, CHW)], out_sem
            ).wait()

    return sc_kernel


def kernel(x, pos_table):
    B, H, W, C = x.shape
    sc_kernel = _make_sc_kernel(B, H, W, C)
    out = sc_kernel(x.reshape(-1), pos_table.reshape(-1))
    return out.reshape(B, H, W, C)


# speedup vs baseline: 1.6618x; 1.6618x over previous
import functools

import jax
import jax.numpy as jnp
from jax import lax
from jax.experimental import pallas as pl
from jax.experimental.pallas import tpu as pltpu
from jax.experimental.pallas import tpu_sc as plsc

MAX_H = 64
MAX_W = 64

NC = 2
NS = 16
L = 16
NRING = 4


def _make_sc_kernel(B, H, W, C):
    mesh = plsc.VectorSubcoreMesh(core_axis_name="c", subcore_axis_name="s")

    @functools.partial(
        pl.kernel,
        mesh=mesh,
        out_type=jax.ShapeDtypeStruct((B, H, W, C), jnp.float32),
        scratch_types=[
            pltpu.VMEM((W, C), jnp.float32),
            pltpu.VMEM((NRING, W, C), jnp.float32),
            pltpu.VMEM((NRING, W, C), jnp.float32),
        ]
        + [pltpu.SemaphoreType.DMA] * (2 * NRING),
    )
    def sc_kernel(x_hbm, pt_hbm, o_hbm, posb, in_b, out_b, *sems):
        in_sems = sems[:NRING]
        out_sems = sems[NRING:]
        h = lax.axis_index("s") * NC + lax.axis_index("c")

        pltpu.sync_copy(pt_hbm.at[pl.ds(h * MAX_W, W)], posb)

        def start_in(b, slot):
            pltpu.make_async_copy(x_hbm.at[b, h], in_b.at[slot], in_sems[slot]).start()

        for s in range(NRING):
            start_in(s, s)

        def add_slab(slot):
            for r in range(W):
                for j in range(C // L):
                    out_b[slot, r, pl.ds(j * L, L)] = (
                        in_b[slot, r, pl.ds(j * L, L)] + posb[r, pl.ds(j * L, L)]
                    )

        def group(g, carry):
            for s in range(NRING):
                b = g * NRING + s
                pltpu.make_async_copy(
                    x_hbm.at[b, h], in_b.at[s], in_sems[s]
                ).wait()

                @pl.when(g >= 1)
                def _():
                    pltpu.make_async_copy(
                        out_b.at[s], o_hbm.at[b - NRING, h], out_sems[s]
                    ).wait()

                add_slab(s)

                pltpu.make_async_copy(
                    out_b.at[s], o_hbm.at[b, h], out_sems[s]
                ).start()

                @pl.when(b + NRING < B)
                def _():
                    start_in(b + NRING, s)

            return carry

        lax.fori_loop(0, B // NRING, group, 0)

        for s in range(NRING):
            pltpu.make_async_copy(
                out_b.at[s], o_hbm.at[0, h], out_sems[s]
            ).wait()

    return sc_kernel


def kernel(x, pos_table):
    B, H, W, C = x.shape
    sc_kernel = _make_sc_kernel(B, H, W, C)
    return sc_kernel(x, pos_table)


# speedup vs baseline: 2.1853x; 1.3151x over previous
import functools

import jax
import jax.numpy as jnp
from jax import lax
from jax.experimental import pallas as pl
from jax.experimental.pallas import tpu as pltpu
from jax.experimental.pallas import tpu_sc as plsc

MAX_H = 64
MAX_W = 64

NC = 2
NS = 16
L = 16
NRING = 4


def _make_sc_kernel(B, H, W, C):
    mesh = plsc.VectorSubcoreMesh(core_axis_name="c", subcore_axis_name="s")

    @functools.partial(
        pl.kernel,
        mesh=mesh,
        out_type=jax.ShapeDtypeStruct((B, H, W, C), jnp.float32),
        scratch_types=[
            pltpu.VMEM((W, C), jnp.float32),
            pltpu.VMEM((NRING, W, C), jnp.float32),
            pltpu.VMEM((NRING, W, C), jnp.float32),
        ]
        + [pltpu.SemaphoreType.DMA] * (2 * NRING),
    )
    def sc_kernel(x_hbm, pt_hbm, o_hbm, posb, in_b, out_b, *sems):
        in_sems = sems[:NRING]
        out_sems = sems[NRING:]
        h = lax.axis_index("s") * NC + lax.axis_index("c")

        pltpu.sync_copy(pt_hbm.at[pl.ds(h * MAX_W, W)], posb)

        def start_in(b, slot):
            pltpu.make_async_copy(x_hbm.at[b, h], in_b.at[slot], in_sems[slot]).start()

        for s in range(NRING):
            start_in(s, s)

        def add_slab(slot):
            @plsc.parallel_loop(0, W, unroll=4)
            def _(r):
                for j in range(C // L):
                    out_b[slot, r, pl.ds(j * L, L)] = (
                        in_b[slot, r, pl.ds(j * L, L)] + posb[r, pl.ds(j * L, L)]
                    )

        def group(g, carry):
            for s in range(NRING):
                b = g * NRING + s
                pltpu.make_async_copy(
                    x_hbm.at[b, h], in_b.at[s], in_sems[s]
                ).wait()

                @pl.when(g >= 1)
                def _():
                    pltpu.make_async_copy(
                        out_b.at[s], o_hbm.at[b - NRING, h], out_sems[s]
                    ).wait()

                add_slab(s)

                pltpu.make_async_copy(
                    out_b.at[s], o_hbm.at[b, h], out_sems[s]
                ).start()

                @pl.when(b + NRING < B)
                def _():
                    start_in(b + NRING, s)

            return carry

        lax.fori_loop(0, B // NRING, group, 0)

        for s in range(NRING):
            pltpu.make_async_copy(
                out_b.at[s], o_hbm.at[0, h], out_sems[s]
            ).wait()

    return sc_kernel


def kernel(x, pos_table):
    B, H, W, C = x.shape
    sc_kernel = _make_sc_kernel(B, H, W, C)
    return sc_kernel(x, pos_table)
